# baseline (speedup 1.0000x reference)
import functools
import jax
import jax.numpy as jnp
from jax import lax
from jax.experimental import pallas as pl
from jax.experimental.pallas import tpu as pltpu
from jax.experimental.pallas import tpu_sc as plsc

_N = 16384
_D = 64
_NL = _D // 16
_NC = 2
_NS = 16
_NW = _NC * _NS
_K = 192
_G = 8


def _spmm_sc(x, cols, rows2, vals, nnz_pad):
    blocks_per_tile = nnz_pad // (_NW * _K)
    assert blocks_per_tile % (2 * _G) == 0
    sb_per_tile = blocks_per_tile // _G
    npair = sb_per_tile // 2
    mesh = plsc.VectorSubcoreMesh(
        core_axis_name="c", subcore_axis_name="s", num_cores=_NC,
        num_subcores=_NS)
    rows_per_tile = _N // _NS

    @functools.partial(
        pl.kernel,
        out_type=(
            jax.ShapeDtypeStruct((_N, _D), jnp.float32),
            jax.ShapeDtypeStruct((_N, _D), jnp.float32),
        ),
        mesh=mesh,
        scratch_types=[
            pltpu.VMEM((_G * _K,), jnp.int32),
            pltpu.VMEM((_G * _K,), jnp.int32),
            pltpu.VMEM((_G, _K), jnp.int32),
            pltpu.VMEM((_G, _K), jnp.int32),
            pltpu.VMEM((_G * _K,), jnp.float32),
            pltpu.VMEM((_G * _K,), jnp.float32),
            pltpu.VMEM((_K, _D), jnp.float32),
            pltpu.VMEM((_K, _D), jnp.float32),
            pltpu.VMEM((_K, _D), jnp.float32),
            pltpu.VMEM((_K, _D), jnp.float32),
            pltpu.VMEM_SHARED((_N, _D), jnp.float32),
            pltpu.SemaphoreType.DMA,
            pltpu.SemaphoreType.DMA,
            pltpu.SemaphoreType.DMA,
            pltpu.SemaphoreType.DMA,
            pltpu.SemaphoreType.DMA,
            pltpu.SemaphoreType.DMA,
            pltpu.SemaphoreType.DMA,
            pltpu.SemaphoreType.DMA,
            pltpu.SemaphoreType.DMA,
            pltpu.SemaphoreType.DMA,
        ],
        compiler_params=pltpu.CompilerParams(use_tc_tiling_on_sc=False),
    )
    def spmm_kernel(x_hbm, cols_hbm, rows_hbm, vals_hbm, p0_hbm, p1_hbm,
                    c0, c1, rw0, rw1, v0, v1, r0, r1, s0, s1, acc_s,
                    ic0, ic1, ir0, ir1, iv0, iv1,
                    gsem0, gsem1, ssem0, ssem1):
        cid = lax.axis_index("c")
        sid = lax.axis_index("s")
        wid = cid * _NS + sid
        colsb = [c0, c1]
        rowsb = [rw0, rw1]
        valsb = [v0, v1]
        icsem = [ic0, ic1]
        irsem = [ir0, ir1]
        ivsem = [iv0, iv1]
        raw = [r0, r1]
        scl = [s0, s1]
        gsem = [gsem0, gsem1]
        ssem = [ssem0, ssem1]

        zero = jnp.zeros((16,), jnp.float32)

        def zrow(j, _):
            for d in range(_NL):
                r0[j, pl.ds(16 * d, 16)] = zero
            return 0

        lax.fori_loop(0, _K, zrow, 0, unroll=8)
        zoff = sid * rows_per_tile
        zrem = rows_per_tile
        while zrem > 0:
            zc = min(zrem, _K)
            pltpu.sync_copy(
                r0.at[pl.ds(0, zc)], acc_s.at[pl.ds(zoff, zc)])
            zoff = zoff + zc
            zrem -= zc
        plsc.subcore_barrier()

        base = wid * (blocks_per_tile * _K)
        base_blk = wid * blocks_per_tile
        last_sb = sb_per_tile - 1

        def issue_idx(set_i, sb):
            off = base + sb * (_G * _K)
            blk0 = base_blk + sb * _G
            return (
                pltpu.async_copy(
                    cols_hbm.at[pl.ds(off, _G * _K)], colsb[set_i],
                    icsem[set_i]),
                pltpu.async_copy(
                    rows_hbm.at[pl.ds(blk0, _G), :], rowsb[set_i],
                    irsem[set_i]),
                pltpu.async_copy(
                    vals_hbm.at[pl.ds(off, _G * _K)], valsb[set_i],
                    ivsem[set_i]),
            )

        splat_idx = [jnp.full((16, 1), r, jnp.int32) for r in range(16)]
        _gd = lax.GatherDimensionNumbers(
            offset_dims=(), collapsed_slice_dims=(0,), start_index_map=(0,))

        def _splat(vv, r):
            return lax.gather(
                vv, splat_idx[r], _gd, (1,),
                mode=lax.GatherScatterMode.PROMISE_IN_BOUNDS)

        def scale(src, dst, vbuf, voff):
            def scale16(t, _):
                j0 = t * 16
                vv = vbuf[pl.ds(voff + j0, 16)]
                for r in range(16):
                    v16 = _splat(vv, r)
                    for d in range(_NL):
                        sl = pl.ds(16 * d, 16)
                        dst[j0 + r, sl] = src[j0 + r, sl] * v16
                return 0

            lax.fori_loop(0, _K // 16, scale16, 0)

        for d in issue_idx(0, 0):
            d.wait()
        pltpu.async_copy(x_hbm.at[c0.at[pl.ds(0, _K)]], r0, gsem0)

        def pair_body(p, _):
            sb0 = 2 * p
            sb1 = sb0 + 1
            gd = [None, None]
            sd = [None, None]
            idxd0 = idxd1 = None
            for i in range(2 * _G):
                half = i // _G
                set_ = half
                b = i % _G
                cur = i & 1
                nxt = cur ^ 1
                if i == 2:
                    idxd1 = issue_idx(1, sb1)
                if i == 10:
                    idxd0 = issue_idx(0, jnp.minimum(sb0 + 2, last_sb))
                if i == _G - 1:
                    for d in idxd1:
                        d.wait()
                if i + 1 < 2 * _G:
                    nhalf = (i + 1) // _G
                    nb = (i + 1) % _G
                    gd[nxt] = pltpu.async_copy(
                        x_hbm.at[colsb[nhalf].at[pl.ds(nb * _K, _K)]],
                        raw[nxt], gsem[nxt])
                if i == 0:
                    pltpu.make_async_copy(
                        x_hbm.at[c0.at[pl.ds(0, _K)]], r0, gsem0).wait()
                else:
                    gd[cur].wait()
                if i >= 2:
                    sd[cur].wait()
                else:
                    @pl.when(p > 0)
                    def _():
                        pltpu.make_async_copy(
                            scl[cur], acc_s.at[rw1.at[_G - 2 + i]],
                            ssem[cur]).wait()
                scale(raw[cur], scl[cur], valsb[set_], b * _K)
                sd[cur] = pltpu.async_copy(
                    scl[cur], acc_s.at[rowsb[set_].at[b]], ssem[cur],
                    add=True)
            for d in idxd0:
                d.wait()
            pltpu.async_copy(x_hbm.at[c0.at[pl.ds(0, _K)]], r0, gsem0)
            return 0

        lax.fori_loop(0, npair, pair_body, 0)

        pltpu.make_async_copy(
            x_hbm.at[c0.at[pl.ds(0, _K)]], r0, gsem0).wait()
        pltpu.make_async_copy(s0, acc_s.at[rw1.at[_G - 2]], ssem0).wait()
        pltpu.make_async_copy(s1, acc_s.at[rw1.at[_G - 1]], ssem1).wait()
        plsc.subcore_barrier()

        row0 = sid * rows_per_tile
        out_ref = [p0_hbm, p1_hbm]
        for c in range(_NC):
            @pl.when(cid == c)
            def _():
                pltpu.sync_copy(
                    acc_s.at[pl.ds(row0, rows_per_tile)],
                    out_ref[c].at[pl.ds(row0, rows_per_tile)])

    return spmm_kernel(x, cols, rows2, vals)


def _combine2(a, b):
    a2 = a.reshape(_N // 2, 128)
    b2 = b.reshape(_N // 2, 128)

    def body(a_ref, b_ref, o_ref):
        o_ref[...] = a_ref[...] + b_ref[...]

    blk = pl.BlockSpec((512, 128), lambda i: (i, 0))
    out = pl.pallas_call(
        body,
        grid=(_N // 2 // 512,),
        in_specs=[blk, blk],
        out_specs=blk,
        out_shape=jax.ShapeDtypeStruct((_N // 2, 128), jnp.float32),
    )(a2, b2)
    return out.reshape(_N, _D)


def _final_mean(x, h1, q0, q1):
    args = [v.reshape(_N // 2, 128) for v in (x, h1, q0, q1)]

    def body(x_ref, h_ref, a_ref, b_ref, o_ref):
        o_ref[...] = (x_ref[...] + h_ref[...] + (a_ref[...] + b_ref[...])
                      ) * jnp.float32(1.0 / 3.0)

    blk = pl.BlockSpec((512, 128), lambda i: (i, 0))
    out = pl.pallas_call(
        body,
        grid=(_N // 2 // 512,),
        in_specs=[blk] * 4,
        out_specs=blk,
        out_shape=jax.ShapeDtypeStruct((_N // 2, 128), jnp.float32),
    )(*args)
    return out.reshape(_N, _D)


@jax.jit
def kernel(x, hg_values, hg_indices):
    nnz = hg_values.shape[0]
    step = _NW * _K * 2 * _G
    nnz_pad = ((nnz + step - 1) // step) * step
    pad = nnz_pad - nnz
    rows2 = jnp.pad(hg_indices[0], (0, pad)).reshape(-1, _K)
    cols = jnp.pad(hg_indices[1], (0, pad))
    vals = jnp.pad(hg_values, (0, pad))

    p0, p1 = _spmm_sc(x, cols, rows2, vals, nnz_pad)
    h1 = _combine2(p0, p1)
    q0, q1 = _spmm_sc(h1, cols, rows2, vals, nnz_pad)
    return _final_mean(x, h1, q0, q1)


# speedup vs baseline: 3.0399x; 3.0399x over previous
import functools
import jax
import jax.numpy as jnp
from jax import lax
from jax.experimental import pallas as pl
from jax.experimental.pallas import tpu as pltpu
from jax.experimental.pallas import tpu_sc as plsc

_N = 16384
_D = 64
_NL = _D // 16
_NC = 2
_NS = 16
_NW = _NC * _NS
_K = 128
_G = 8


def _spmm_sc(x, cols, rows2, vals, nnz_pad):
    blocks_per_tile = nnz_pad // (_NW * _K)
    assert blocks_per_tile % (2 * _G) == 0
    sb_per_tile = blocks_per_tile // _G
    npair = sb_per_tile // 2
    mesh = plsc.VectorSubcoreMesh(
        core_axis_name="c", subcore_axis_name="s", num_cores=_NC,
        num_subcores=_NS)
    rows_per_tile = _N // _NS

    @functools.partial(
        pl.kernel,
        out_type=(
            jax.ShapeDtypeStruct((_N, _D), jnp.float32),
            jax.ShapeDtypeStruct((_N, _D), jnp.float32),
        ),
        mesh=mesh,
        scratch_types=[
            pltpu.VMEM((_G * _K,), jnp.int32),
            pltpu.VMEM((_G * _K,), jnp.int32),
            pltpu.VMEM((_G, _K), jnp.int32),
            pltpu.VMEM((_G, _K), jnp.int32),
            pltpu.VMEM((_G * _K,), jnp.float32),
            pltpu.VMEM((_G * _K,), jnp.float32),
            pltpu.VMEM((_K, _D), jnp.float32),
            pltpu.VMEM((_K, _D), jnp.float32),
            pltpu.VMEM((_K, _D), jnp.float32),
            pltpu.VMEM((_K, _D), jnp.float32),
            pltpu.VMEM_SHARED((_N, _D), jnp.float32),
            pltpu.SemaphoreType.DMA,
            pltpu.SemaphoreType.DMA,
            pltpu.SemaphoreType.DMA,
            pltpu.SemaphoreType.DMA,
            pltpu.SemaphoreType.DMA,
            pltpu.SemaphoreType.DMA,
            pltpu.SemaphoreType.DMA,
            pltpu.SemaphoreType.DMA,
            pltpu.SemaphoreType.DMA,
            pltpu.SemaphoreType.DMA,
        ],
        compiler_params=pltpu.CompilerParams(use_tc_tiling_on_sc=False),
    )
    def spmm_kernel(x_hbm, cols_hbm, rows_hbm, vals_hbm, p0_hbm, p1_hbm,
                    c0, c1, rw0, rw1, v0, v1, r0, r1, s0, s1, acc_s,
                    ic0, ic1, ir0, ir1, iv0, iv1,
                    gsem0, gsem1, ssem0, ssem1):
        cid = lax.axis_index("c")
        sid = lax.axis_index("s")
        wid = cid * _NS + sid
        colsb = [c0, c1]
        rowsb = [rw0, rw1]
        valsb = [v0, v1]
        icsem = [ic0, ic1]
        irsem = [ir0, ir1]
        ivsem = [iv0, iv1]
        raw = [r0, r1]
        scl = [s0, s1]
        gsem = [gsem0, gsem1]
        ssem = [ssem0, ssem1]

        zero = jnp.zeros((16,), jnp.float32)

        def zrow(j, _):
            for d in range(_NL):
                r0[j, pl.ds(16 * d, 16)] = zero
            return 0

        lax.fori_loop(0, _K, zrow, 0, unroll=8)
        for i in range(rows_per_tile // _K):
            pltpu.sync_copy(
                r0, acc_s.at[pl.ds(sid * rows_per_tile + i * _K, _K)])
        plsc.subcore_barrier()

        base = wid * (blocks_per_tile * _K)
        base_blk = wid * blocks_per_tile
        last_sb = sb_per_tile - 1

        def issue_idx(set_i, sb):
            off = base + sb * (_G * _K)
            blk0 = base_blk + sb * _G
            return (
                pltpu.async_copy(
                    cols_hbm.at[pl.ds(off, _G * _K)], colsb[set_i],
                    icsem[set_i]),
                pltpu.async_copy(
                    rows_hbm.at[pl.ds(blk0, _G), :], rowsb[set_i],
                    irsem[set_i]),
                pltpu.async_copy(
                    vals_hbm.at[pl.ds(off, _G * _K)], valsb[set_i],
                    ivsem[set_i]),
            )

        splat_idx = [jnp.full((16, 1), r, jnp.int32) for r in range(16)]
        _gd = lax.GatherDimensionNumbers(
            offset_dims=(), collapsed_slice_dims=(0,), start_index_map=(0,))

        def _splat(vv, r):
            return lax.gather(
                vv, splat_idx[r], _gd, (1,),
                mode=lax.GatherScatterMode.PROMISE_IN_BOUNDS)

        def scale(src, dst, vbuf, voff):
            def scale16(t, _):
                j0 = t * 16
                vv = vbuf[pl.ds(voff + j0, 16)]
                for r in range(16):
                    v16 = _splat(vv, r)
                    for d in range(_NL):
                        sl = pl.ds(16 * d, 16)
                        dst[j0 + r, sl] = src[j0 + r, sl] * v16
                return 0

            lax.fori_loop(0, _K // 16, scale16, 0)

        for d in issue_idx(0, 0):
            d.wait()
        pltpu.async_copy(x_hbm.at[c0.at[pl.ds(0, _K)]], r0, gsem0)

        def pair_body(p, _):
            sb0 = 2 * p
            sb1 = sb0 + 1
            gd = [None, None]
            sd = [None, None]
            idxd0 = idxd1 = None
            for i in range(2 * _G):
                half = i // _G
                set_ = half
                b = i % _G
                cur = i & 1
                nxt = cur ^ 1
                if i == 2:
                    idxd1 = issue_idx(1, sb1)
                if i == 10:
                    idxd0 = issue_idx(0, jnp.minimum(sb0 + 2, last_sb))
                if i == _G - 1:
                    for d in idxd1:
                        d.wait()
                if i + 1 < 2 * _G:
                    nhalf = (i + 1) // _G
                    nb = (i + 1) % _G
                    gd[nxt] = pltpu.async_copy(
                        x_hbm.at[colsb[nhalf].at[pl.ds(nb * _K, _K)]],
                        raw[nxt], gsem[nxt])
                if i == 0:
                    pltpu.make_async_copy(
                        x_hbm.at[c0.at[pl.ds(0, _K)]], r0, gsem0).wait()
                else:
                    gd[cur].wait()
                if i >= 2:
                    sd[cur].wait()
                else:
                    @pl.when(p > 0)
                    def _():
                        pltpu.make_async_copy(
                            scl[cur], acc_s.at[rw1.at[_G - 2 + i]],
                            ssem[cur]).wait()
                scale(raw[cur], scl[cur], valsb[set_], b * _K)
                sd[cur] = pltpu.async_copy(
                    scl[cur], acc_s.at[rowsb[set_].at[b]], ssem[cur],
                    add=True)
            for d in idxd0:
                d.wait()
            pltpu.async_copy(x_hbm.at[c0.at[pl.ds(0, _K)]], r0, gsem0)
            return 0

        lax.fori_loop(0, npair, pair_body, 0)

        pltpu.make_async_copy(
            x_hbm.at[c0.at[pl.ds(0, _K)]], r0, gsem0).wait()
        pltpu.make_async_copy(s0, acc_s.at[rw1.at[_G - 2]], ssem0).wait()
        pltpu.make_async_copy(s1, acc_s.at[rw1.at[_G - 1]], ssem1).wait()
        plsc.subcore_barrier()

        row0 = sid * rows_per_tile
        out_ref = [p0_hbm, p1_hbm]
        for c in range(_NC):
            @pl.when(cid == c)
            def _():
                pltpu.sync_copy(
                    acc_s.at[pl.ds(row0, rows_per_tile)],
                    out_ref[c].at[pl.ds(row0, rows_per_tile)])

    return spmm_kernel(x, cols, rows2, vals)


def _combine2(a, b):
    a2 = a.reshape(_N // 2, 128)
    b2 = b.reshape(_N // 2, 128)

    def body(a_ref, b_ref, o_ref):
        o_ref[...] = a_ref[...] + b_ref[...]

    blk = pl.BlockSpec((512, 128), lambda i: (i, 0))
    out = pl.pallas_call(
        body,
        grid=(_N // 2 // 512,),
        in_specs=[blk, blk],
        out_specs=blk,
        out_shape=jax.ShapeDtypeStruct((_N // 2, 128), jnp.float32),
    )(a2, b2)
    return out.reshape(_N, _D)


def _final_mean(x, h1, q0, q1):
    args = [v.reshape(_N // 2, 128) for v in (x, h1, q0, q1)]

    def body(x_ref, h_ref, a_ref, b_ref, o_ref):
        o_ref[...] = (x_ref[...] + h_ref[...] + (a_ref[...] + b_ref[...])
                      ) * jnp.float32(1.0 / 3.0)

    blk = pl.BlockSpec((512, 128), lambda i: (i, 0))
    out = pl.pallas_call(
        body,
        grid=(_N // 2 // 512,),
        in_specs=[blk] * 4,
        out_specs=blk,
        out_shape=jax.ShapeDtypeStruct((_N // 2, 128), jnp.float32),
    )(*args)
    return out.reshape(_N, _D)


@jax.jit
def kernel(x, hg_values, hg_indices):
    nnz = hg_values.shape[0]
    step = _NW * _K * 2 * _G
    nnz_pad = ((nnz + step - 1) // step) * step
    pad = nnz_pad - nnz
    rows2 = jnp.pad(hg_indices[0], (0, pad)).reshape(-1, _K)
    cols = jnp.pad(hg_indices[1], (0, pad))
    vals = jnp.pad(hg_values, (0, pad))

    p0, p1 = _spmm_sc(x, cols, rows2, vals, nnz_pad)
    h1 = _combine2(p0, p1)
    q0, q1 = _spmm_sc(h1, cols, rows2, vals, nnz_pad)
    return _final_mean(x, h1, q0, q1)
